# baseline (speedup 1.0000x reference)
import functools

import jax
import jax.numpy as jnp
from jax import lax
from jax.experimental import pallas as pl
from jax.experimental.pallas import tpu as pltpu
from jax.experimental.pallas import tpu_sc as plsc

BATCH = 16384
D = 64
L = 16
NROWS = 1000000
BLK = 128
LAST_TC = (NROWS // BLK) - 1
EDGE0 = (NROWS // BLK) * BLK


SC_N = 12288
TC_N = BATCH - SC_N
EPG = 16


def _build_sc_call():
    mesh = plsc.VectorSubcoreMesh(core_axis_name="c", subcore_axis_name="s")
    nc, ns = mesh.num_cores, mesh.num_subcores
    b_per_w = SC_N // (nc * ns)
    n_pairs = b_per_w // 2

    @functools.partial(
        pl.kernel,
        out_type=jax.ShapeDtypeStruct((SC_N,), jnp.float32),
        mesh=mesh,
        scratch_types=[
            pltpu.VMEM((b_per_w + L,), jnp.int32),
            pltpu.VMEM((b_per_w + L,), jnp.int32),
            pltpu.VMEM((5, D, BLK), jnp.float32),
            pltpu.VMEM((5, D, BLK), jnp.float32),
            pltpu.VMEM((b_per_w,), jnp.float32),
            pltpu.VMEM((D,), jnp.float32),
            pltpu.VMEM((L,), jnp.float32),
            pltpu.SemaphoreType.DMA,
            pltpu.SemaphoreType.DMA,
            pltpu.SemaphoreType.DMA,
            pltpu.SemaphoreType.DMA,
        ],
        compiler_params=pltpu.CompilerParams(
            use_tc_tiling_on_sc=True, needs_layout_passes=False),
    )
    def neumf_kernel(uidx_hbm, iidx_hbm, ut_hbm, it_hbm, uedge_hbm, iedge_hbm,
                     w_hbm, b_hbm, out_hbm, idx_u, idx_i, u_all, i_all, out_v,
                     w_v, b_v, sem0, sem1, sem2, sem3):
        wid = lax.axis_index("s") * nc + lax.axis_index("c")
        base = wid * b_per_w
        pltpu.sync_copy(uidx_hbm.at[pl.ds(base, b_per_w)],
                        idx_u.at[pl.ds(0, b_per_w)])
        pltpu.sync_copy(iidx_hbm.at[pl.ds(base, b_per_w)],
                        idx_i.at[pl.ds(0, b_per_w)])

        def sidx(ref, e):
            return ref[pl.ds(e, L)][0]
        pltpu.sync_copy(w_hbm, w_v)
        pltpu.sync_copy(b_hbm, b_v)
        pltpu.sync_copy(uedge_hbm, u_all.at[4])
        pltpu.sync_copy(iedge_hbm, i_all.at[4])

        sems = (sem0, sem1, sem2, sem3)
        lane = lax.iota(jnp.int32, L)
        perms = [jnp.bitwise_xor(lane, d) for d in (1, 2, 4, 8)]
        dnums = lax.GatherDimensionNumbers(
            offset_dims=(), collapsed_slice_dims=(0,), start_index_map=(0,))

        def lane_sum(s):
            for p in perms:
                s = s + lax.gather(s, p[:, None], dnums, (1,),
                                   mode=lax.GatherScatterMode.PROMISE_IN_BOUNDS)
            return s

        w_chunks = [w_v[pl.ds(c * L, L)] for c in range(D // L)]
        bd = b_v[...]

        def fire(e, par):
            tcu = jnp.minimum(sidx(idx_u, e) >> 7, LAST_TC)
            tci = jnp.minimum(sidx(idx_i, e) >> 7, LAST_TC)
            pltpu.async_copy(ut_hbm.at[:, pl.ds(tcu * BLK, BLK)],
                             u_all.at[par], sems[par])
            pltpu.async_copy(it_hbm.at[:, pl.ds(tci * BLK, BLK)],
                             i_all.at[par], sems[par])

        def drain(par):
            dummy = ut_hbm.at[:, pl.ds(0, BLK)]
            pltpu.make_async_copy(dummy, u_all.at[par], sems[par]).wait()
            pltpu.make_async_copy(dummy, i_all.at[par], sems[par]).wait()

        def element_value(e, par):
            ru = sidx(idx_u, e)
            ri = sidx(idx_i, e)
            srcu = jnp.full((L,), jnp.where(ru >= EDGE0, 4, par), jnp.int32)
            srci = jnp.full((L,), jnp.where(ri >= EDGE0, 4, par), jnp.int32)
            rcu = jnp.full((L,), ru & (BLK - 1), jnp.int32)
            rci = jnp.full((L,), ri & (BLK - 1), jnp.int32)
            s = bd
            for c in range(D // L):
                fv = c * L + lane
                gu = plsc.load_gather(u_all, [srcu, fv, rcu])
                gi = plsc.load_gather(i_all, [srci, fv, rci])
                s = s + gu * gi * w_chunks[c]
            return lane_sum(s)

        NBUF = 4
        for par in range(NBUF):
            fire(par, par)

        def quad_body(t, acc):
            e0 = NBUF * t
            for par in range(NBUF):
                e = e0 + par
                drain(par)
                v = element_value(e, par)
                fire(jnp.minimum(e + NBUF, b_per_w - 1), par)
                acc = jnp.where(lane == (e & 15), v, acc)

            @pl.when((t & 3) == 3)
            def _():
                out_v[pl.ds((t >> 2) * L, L)] = acc

            return jnp.where(jnp.full((L,), (t & 3) == 3), jnp.zeros_like(acc),
                             acc)

        lax.fori_loop(0, b_per_w // NBUF, quad_body,
                      jnp.zeros((L,), jnp.float32))
        for par in range(NBUF):
            drain(par)

        pltpu.sync_copy(out_v, out_hbm.at[pl.ds(base, b_per_w)])

    return neumf_kernel


def _build_tc_call():
    n_steps = TC_N // EPG

    def umap(k):
        return lambda e, tcu, tci, rcu, rci, fu, fi: (0, tcu[e * EPG + k])

    def imap(k):
        return lambda e, tcu, tci, rcu, rci, fu, fi: (0, tci[e * EPG + k])

    const2 = lambda e, *refs: (0, 0)
    in_specs = ([pl.BlockSpec((D, BLK), umap(k)) for k in range(EPG)]
                + [pl.BlockSpec((D, BLK), imap(k)) for k in range(EPG)]
                + [pl.BlockSpec((D, BLK), const2),
                   pl.BlockSpec((D, BLK), const2),
                   pl.BlockSpec((D, 1), const2)])
    grid_spec = pltpu.PrefetchScalarGridSpec(
        num_scalar_prefetch=6,
        grid=(n_steps,),
        in_specs=in_specs,
        out_specs=pl.BlockSpec((1, 1, BLK), lambda e, *refs: (e, 0, 0)),
    )

    def tc_kernel(tcu, tci, rcu, rci, fu, fi, *refs):
        u_refs = refs[:EPG]
        i_refs = refs[EPG:2 * EPG]
        ue_ref, ie_ref, w_ref, o_ref = refs[2 * EPG:]
        e = pl.program_id(0)
        w = w_ref[...]
        vals = []
        for k in range(EPG):
            g = e * EPG + k
            ublk = jnp.where(fu[g] == 1, ue_ref[...], u_refs[k][...])
            iblk = jnp.where(fi[g] == 1, ie_ref[...], i_refs[k][...])
            su = (BLK - rcu[g]) & (BLK - 1)
            si = (BLK - rci[g]) & (BLK - 1)
            ucol = pltpu.roll(ublk, su, 1)[:, 0:1]
            icol = pltpu.roll(iblk, si, 1)[:, 0:1]
            vals.append(jnp.sum(ucol * icol * w))
        o_ref[...] = jnp.pad(jnp.stack(vals), (0, BLK - EPG)).reshape(1, 1, BLK)

    return pl.pallas_call(
        tc_kernel,
        grid_spec=grid_spec,
        out_shape=jax.ShapeDtypeStruct((n_steps, 1, BLK), jnp.float32),
        compiler_params=pltpu.CompilerParams(
            dimension_semantics=("arbitrary",)),
    )


def kernel(users_index, items_index, user_mf_emb, item_mf_emb, W_pred, b_pred):
    ut = user_mf_emb.T
    it = item_mf_emb.T
    uedge = jnp.pad(user_mf_emb[EDGE0:].T, ((0, 0), (0, BLK - (NROWS - EDGE0))))
    iedge = jnp.pad(item_mf_emb[EDGE0:].T, ((0, 0), (0, BLK - (NROWS - EDGE0))))
    w_flat = W_pred.reshape(D)
    b_lane = jnp.full((L,), b_pred[0] / L, dtype=jnp.float32)
    ui = users_index.astype(jnp.int32)
    ii = items_index.astype(jnp.int32)

    sc_call = _build_sc_call()
    out_sc = sc_call(ui, ii, ut, it, uedge, iedge, w_flat, b_lane)

    ut_tc = ui[SC_N:]
    it_tc = ii[SC_N:]
    tcu = jnp.minimum(ut_tc >> 7, LAST_TC)
    tci = jnp.minimum(it_tc >> 7, LAST_TC)
    rcu = ut_tc & (BLK - 1)
    rci = it_tc & (BLK - 1)
    fu = (ut_tc >= EDGE0).astype(jnp.int32)
    fi = (it_tc >= EDGE0).astype(jnp.int32)
    tc_call = _build_tc_call()
    out_tc2 = tc_call(tcu, tci, rcu, rci, fu, fi,
                      *([ut] * EPG + [it] * EPG + [uedge, iedge, W_pred]))
    out_tc = out_tc2[:, 0, :EPG].reshape(TC_N) + b_pred[0]

    return jnp.concatenate([out_sc, out_tc]).reshape(BATCH, 1)


# speedup vs baseline: 1.1526x; 1.1526x over previous
import functools

import jax
import jax.numpy as jnp
from jax import lax
from jax.experimental import pallas as pl
from jax.experimental.pallas import tpu as pltpu
from jax.experimental.pallas import tpu_sc as plsc

BATCH = 16384
D = 64
L = 16
NROWS = 1000000
BLK = 128
LAST_TC = (NROWS // BLK) - 1
EDGE0 = (NROWS // BLK) * BLK


def _build_sc_call():
    mesh = plsc.VectorSubcoreMesh(core_axis_name="c", subcore_axis_name="s")
    nc, ns = mesh.num_cores, mesh.num_subcores
    b_per_w = BATCH // (nc * ns)
    n_pairs = b_per_w // 2

    @functools.partial(
        pl.kernel,
        out_type=jax.ShapeDtypeStruct((BATCH,), jnp.float32),
        mesh=mesh,
        scratch_types=[
            pltpu.VMEM((b_per_w + L,), jnp.int32),
            pltpu.VMEM((b_per_w + L,), jnp.int32),
            pltpu.VMEM((5, D, BLK), jnp.float32),
            pltpu.VMEM((5, D, BLK), jnp.float32),
            pltpu.VMEM((b_per_w,), jnp.float32),
            pltpu.VMEM((D,), jnp.float32),
            pltpu.VMEM((L,), jnp.float32),
            pltpu.SemaphoreType.DMA,
            pltpu.SemaphoreType.DMA,
            pltpu.SemaphoreType.DMA,
            pltpu.SemaphoreType.DMA,
        ],
        compiler_params=pltpu.CompilerParams(
            use_tc_tiling_on_sc=True, needs_layout_passes=False),
    )
    def neumf_kernel(uidx_hbm, iidx_hbm, ut_hbm, it_hbm, uedge_hbm, iedge_hbm,
                     w_hbm, b_hbm, out_hbm, idx_u, idx_i, u_all, i_all, out_v,
                     w_v, b_v, sem0, sem1, sem2, sem3):
        wid = lax.axis_index("s") * nc + lax.axis_index("c")
        base = wid * b_per_w
        pltpu.sync_copy(uidx_hbm.at[pl.ds(base, b_per_w)],
                        idx_u.at[pl.ds(0, b_per_w)])
        pltpu.sync_copy(iidx_hbm.at[pl.ds(base, b_per_w)],
                        idx_i.at[pl.ds(0, b_per_w)])

        def sidx(ref, e):
            return ref[pl.ds(e, L)][0]
        pltpu.sync_copy(w_hbm, w_v)
        pltpu.sync_copy(b_hbm, b_v)
        pltpu.sync_copy(uedge_hbm, u_all.at[4])
        pltpu.sync_copy(iedge_hbm, i_all.at[4])

        sems = (sem0, sem1, sem2, sem3)
        lane = lax.iota(jnp.int32, L)
        perms = [jnp.bitwise_xor(lane, d) for d in (1, 2, 4, 8)]
        dnums = lax.GatherDimensionNumbers(
            offset_dims=(), collapsed_slice_dims=(0,), start_index_map=(0,))

        def lane_sum(s):
            for p in perms:
                s = s + lax.gather(s, p[:, None], dnums, (1,),
                                   mode=lax.GatherScatterMode.PROMISE_IN_BOUNDS)
            return s

        w_chunks = [w_v[pl.ds(c * L, L)] for c in range(D // L)]
        bd = b_v[...]

        def fire(e, par):
            tcu = jnp.minimum(sidx(idx_u, e) >> 7, LAST_TC)
            tci = jnp.minimum(sidx(idx_i, e) >> 7, LAST_TC)
            pltpu.async_copy(ut_hbm.at[:, pl.ds(tcu * BLK, BLK)],
                             u_all.at[par], sems[par])
            pltpu.async_copy(it_hbm.at[:, pl.ds(tci * BLK, BLK)],
                             i_all.at[par], sems[par])

        def drain(par):
            dummy = ut_hbm.at[:, pl.ds(0, BLK)]
            pltpu.make_async_copy(dummy, u_all.at[par], sems[par]).wait()
            pltpu.make_async_copy(dummy, i_all.at[par], sems[par]).wait()

        def element_value(e, par):
            ru = sidx(idx_u, e)
            ri = sidx(idx_i, e)
            srcu = jnp.full((L,), jnp.where(ru >= EDGE0, 4, par), jnp.int32)
            srci = jnp.full((L,), jnp.where(ri >= EDGE0, 4, par), jnp.int32)
            rcu = jnp.full((L,), ru & (BLK - 1), jnp.int32)
            rci = jnp.full((L,), ri & (BLK - 1), jnp.int32)
            s = bd
            for c in range(D // L):
                fv = c * L + lane
                gu = plsc.load_gather(u_all, [srcu, fv, rcu])
                gi = plsc.load_gather(i_all, [srci, fv, rci])
                s = s + gu * gi * w_chunks[c]
            return lane_sum(s)

        NBUF = 4
        for par in range(NBUF):
            fire(par, par)

        def quad_body(t, acc):
            e0 = NBUF * t
            for par in range(NBUF):
                e = e0 + par
                drain(par)
                v = element_value(e, par)
                fire(jnp.minimum(e + NBUF, b_per_w - 1), par)
                acc = jnp.where(lane == (e & 15), v, acc)

            @pl.when((t & 3) == 3)
            def _():
                out_v[pl.ds((t >> 2) * L, L)] = acc

            return jnp.where(jnp.full((L,), (t & 3) == 3), jnp.zeros_like(acc),
                             acc)

        lax.fori_loop(0, b_per_w // NBUF, quad_body,
                      jnp.zeros((L,), jnp.float32))
        for par in range(NBUF):
            drain(par)

        pltpu.sync_copy(out_v, out_hbm.at[pl.ds(base, b_per_w)])

    return neumf_kernel


def kernel(users_index, items_index, user_mf_emb, item_mf_emb, W_pred, b_pred):
    ut = user_mf_emb.T
    it = item_mf_emb.T
    uedge = jnp.pad(user_mf_emb[EDGE0:].T, ((0, 0), (0, BLK - (NROWS - EDGE0))))
    iedge = jnp.pad(item_mf_emb[EDGE0:].T, ((0, 0), (0, BLK - (NROWS - EDGE0))))
    w_flat = W_pred.reshape(D)
    b_lane = jnp.full((L,), b_pred[0] / L, dtype=jnp.float32)
    call = _build_sc_call()
    out = call(users_index.astype(jnp.int32), items_index.astype(jnp.int32),
               ut, it, uedge, iedge, w_flat, b_lane)
    return out.reshape(BATCH, 1)


# speedup vs baseline: 1.2029x; 1.0436x over previous
import functools

import jax
import jax.numpy as jnp
from jax import lax
from jax.experimental import pallas as pl
from jax.experimental.pallas import tpu as pltpu
from jax.experimental.pallas import tpu_sc as plsc

BATCH = 16384
D = 64
L = 16
NROWS = 1000000
BLK = 128
LAST_TC = (NROWS // BLK) - 1
EDGE0 = (NROWS // BLK) * BLK


def _build_sc_call():
    mesh = plsc.VectorSubcoreMesh(core_axis_name="c", subcore_axis_name="s")
    nc, ns = mesh.num_cores, mesh.num_subcores
    b_per_w = BATCH // (nc * ns)
    n_pairs = b_per_w // 2

    @functools.partial(
        pl.kernel,
        out_type=jax.ShapeDtypeStruct((BATCH,), jnp.float32),
        mesh=mesh,
        scratch_types=[
            pltpu.VMEM((b_per_w + L,), jnp.int32),
            pltpu.VMEM((b_per_w + L,), jnp.int32),
            pltpu.VMEM((6, D, BLK), jnp.float32),
            pltpu.VMEM((6, D, BLK), jnp.float32),
            pltpu.VMEM((b_per_w,), jnp.float32),
            pltpu.VMEM((D,), jnp.float32),
            pltpu.VMEM((L,), jnp.float32),
            pltpu.SemaphoreType.DMA,
            pltpu.SemaphoreType.DMA,
            pltpu.SemaphoreType.DMA,
            pltpu.SemaphoreType.DMA,
            pltpu.SemaphoreType.DMA,
        ],
        compiler_params=pltpu.CompilerParams(
            use_tc_tiling_on_sc=True, needs_layout_passes=False),
    )
    def neumf_kernel(uidx_hbm, iidx_hbm, ut_hbm, it_hbm, uedge_hbm, iedge_hbm,
                     w_hbm, b_hbm, out_hbm, idx_u, idx_i, u_all, i_all, out_v,
                     w_v, b_v, sem0, sem1, sem2, sem3, sem4):
        wid = lax.axis_index("s") * nc + lax.axis_index("c")
        base = wid * b_per_w
        pltpu.sync_copy(uidx_hbm.at[pl.ds(base, b_per_w)],
                        idx_u.at[pl.ds(0, b_per_w)])
        pltpu.sync_copy(iidx_hbm.at[pl.ds(base, b_per_w)],
                        idx_i.at[pl.ds(0, b_per_w)])

        def sidx(ref, e):
            return ref[pl.ds(e, L)][0]
        pltpu.sync_copy(w_hbm, w_v)
        pltpu.sync_copy(b_hbm, b_v)
        pltpu.sync_copy(uedge_hbm, u_all.at[5])
        pltpu.sync_copy(iedge_hbm, i_all.at[5])

        sems = (sem0, sem1, sem2, sem3, sem4)
        lane = lax.iota(jnp.int32, L)
        perms = [jnp.bitwise_xor(lane, d) for d in (1, 2, 4, 8)]
        dnums = lax.GatherDimensionNumbers(
            offset_dims=(), collapsed_slice_dims=(0,), start_index_map=(0,))

        def lane_sum(s):
            for p in perms:
                s = s + lax.gather(s, p[:, None], dnums, (1,),
                                   mode=lax.GatherScatterMode.PROMISE_IN_BOUNDS)
            return s

        w_chunks = [w_v[pl.ds(c * L, L)] for c in range(D // L)]
        bd = b_v[...]

        def fire(e, par):
            tcu = jnp.minimum(sidx(idx_u, e) >> 7, LAST_TC)
            tci = jnp.minimum(sidx(idx_i, e) >> 7, LAST_TC)
            pltpu.async_copy(ut_hbm.at[:, pl.ds(tcu * BLK, BLK)],
                             u_all.at[par], sems[par])
            pltpu.async_copy(it_hbm.at[:, pl.ds(tci * BLK, BLK)],
                             i_all.at[par], sems[par])

        def drain(par):
            dummy = ut_hbm.at[:, pl.ds(0, BLK)]
            pltpu.make_async_copy(dummy, u_all.at[par], sems[par]).wait()
            pltpu.make_async_copy(dummy, i_all.at[par], sems[par]).wait()

        def element_value(e, par):
            ru = sidx(idx_u, e)
            ri = sidx(idx_i, e)
            srcu = jnp.full((L,), jnp.where(ru >= EDGE0, 5, par), jnp.int32)
            srci = jnp.full((L,), jnp.where(ri >= EDGE0, 5, par), jnp.int32)
            rcu = jnp.full((L,), ru & (BLK - 1), jnp.int32)
            rci = jnp.full((L,), ri & (BLK - 1), jnp.int32)
            s = bd
            for c in range(D // L):
                fv = c * L + lane
                gu = plsc.load_gather(u_all, [srcu, fv, rcu])
                gi = plsc.load_gather(i_all, [srci, fv, rci])
                s = s + gu * gi * w_chunks[c]
            return lane_sum(s)

        NBUF = 5
        n_iters = (b_per_w + NBUF - 1) // NBUF
        for par in range(NBUF):
            fire(par, par)

        def ring_body(t, carry):
            e0 = NBUF * t
            for par in range(NBUF):
                e = jnp.minimum(e0 + par, b_per_w - 1)
                drain(par)
                v = element_value(e, par)
                fire(jnp.minimum(e0 + par + NBUF, b_per_w - 1), par)
                plsc.store_scatter(out_v, [jnp.full((L,), e, jnp.int32)], v,
                                   mask=lane == 0)
            return carry

        lax.fori_loop(0, n_iters, ring_body, 0)
        for par in range(NBUF):
            drain(par)

        pltpu.sync_copy(out_v, out_hbm.at[pl.ds(base, b_per_w)])

    return neumf_kernel


def kernel(users_index, items_index, user_mf_emb, item_mf_emb, W_pred, b_pred):
    ut = user_mf_emb.T
    it = item_mf_emb.T
    uedge = jnp.pad(user_mf_emb[EDGE0:].T, ((0, 0), (0, BLK - (NROWS - EDGE0))))
    iedge = jnp.pad(item_mf_emb[EDGE0:].T, ((0, 0), (0, BLK - (NROWS - EDGE0))))
    w_flat = W_pred.reshape(D)
    b_lane = jnp.full((L,), b_pred[0] / L, dtype=jnp.float32)
    call = _build_sc_call()
    out = call(users_index.astype(jnp.int32), items_index.astype(jnp.int32),
               ut, it, uedge, iedge, w_flat, b_lane)
    return out.reshape(BATCH, 1)


# speedup vs baseline: 1.2566x; 1.0446x over previous
import functools

import jax
import jax.numpy as jnp
from jax import lax
from jax.experimental import pallas as pl
from jax.experimental.pallas import tpu as pltpu
from jax.experimental.pallas import tpu_sc as plsc

BATCH = 16384
D = 64
L = 16
NROWS = 1000000
BLK = 128
LAST_TC = (NROWS // BLK) - 1
EDGE0 = (NROWS // BLK) * BLK


def _build_sc_call():
    mesh = plsc.VectorSubcoreMesh(core_axis_name="c", subcore_axis_name="s")
    nc, ns = mesh.num_cores, mesh.num_subcores
    b_per_w = BATCH // (nc * ns)
    n_pairs = b_per_w // 2

    @functools.partial(
        pl.kernel,
        out_type=jax.ShapeDtypeStruct((BATCH,), jnp.float32),
        mesh=mesh,
        scratch_types=[
            pltpu.VMEM((b_per_w + L,), jnp.int32),
            pltpu.VMEM((b_per_w + L,), jnp.int32),
            pltpu.VMEM((7, D, BLK), jnp.float32),
            pltpu.VMEM((7, D, BLK), jnp.float32),
            pltpu.VMEM((b_per_w,), jnp.float32),
            pltpu.VMEM((D,), jnp.float32),
            pltpu.VMEM((L,), jnp.float32),
            pltpu.SemaphoreType.DMA,
            pltpu.SemaphoreType.DMA,
            pltpu.SemaphoreType.DMA,
            pltpu.SemaphoreType.DMA,
            pltpu.SemaphoreType.DMA,
            pltpu.SemaphoreType.DMA,
        ],
        compiler_params=pltpu.CompilerParams(
            use_tc_tiling_on_sc=True, needs_layout_passes=False),
    )
    def neumf_kernel(uidx_hbm, iidx_hbm, ut_hbm, it_hbm, uedge_hbm, iedge_hbm,
                     w_hbm, b_hbm, out_hbm, idx_u, idx_i, u_all, i_all, out_v,
                     w_v, b_v, sem0, sem1, sem2, sem3, sem4, sem5):
        wid = lax.axis_index("s") * nc + lax.axis_index("c")
        base = wid * b_per_w
        pltpu.sync_copy(uidx_hbm.at[pl.ds(base, b_per_w)],
                        idx_u.at[pl.ds(0, b_per_w)])
        pltpu.sync_copy(iidx_hbm.at[pl.ds(base, b_per_w)],
                        idx_i.at[pl.ds(0, b_per_w)])

        def sidx(ref, e):
            return ref[pl.ds(e, L)][0]
        pltpu.sync_copy(w_hbm, w_v)
        pltpu.sync_copy(b_hbm, b_v)
        pltpu.sync_copy(uedge_hbm, u_all.at[6])
        pltpu.sync_copy(iedge_hbm, i_all.at[6])

        sems = (sem0, sem1, sem2, sem3, sem4, sem5)
        lane = lax.iota(jnp.int32, L)
        perms = [jnp.bitwise_xor(lane, d) for d in (1, 2, 4, 8)]
        dnums = lax.GatherDimensionNumbers(
            offset_dims=(), collapsed_slice_dims=(0,), start_index_map=(0,))

        def lane_sum(s):
            for p in perms:
                s = s + lax.gather(s, p[:, None], dnums, (1,),
                                   mode=lax.GatherScatterMode.PROMISE_IN_BOUNDS)
            return s

        w_chunks = [w_v[pl.ds(c * L, L)] for c in range(D // L)]
        bd = b_v[...]

        def fire(e, par):
            tcu = jnp.minimum(sidx(idx_u, e) >> 7, LAST_TC)
            tci = jnp.minimum(sidx(idx_i, e) >> 7, LAST_TC)
            pltpu.async_copy(ut_hbm.at[:, pl.ds(tcu * BLK, BLK)],
                             u_all.at[par], sems[par])
            pltpu.async_copy(it_hbm.at[:, pl.ds(tci * BLK, BLK)],
                             i_all.at[par], sems[par])

        def drain(par):
            dummy = ut_hbm.at[:, pl.ds(0, BLK)]
            pltpu.make_async_copy(dummy, u_all.at[par], sems[par]).wait()
            pltpu.make_async_copy(dummy, i_all.at[par], sems[par]).wait()

        def element_value(e, par):
            ru = sidx(idx_u, e)
            ri = sidx(idx_i, e)
            srcu = jnp.full((L,), jnp.where(ru >= EDGE0, 6, par), jnp.int32)
            srci = jnp.full((L,), jnp.where(ri >= EDGE0, 6, par), jnp.int32)
            rcu = jnp.full((L,), ru & (BLK - 1), jnp.int32)
            rci = jnp.full((L,), ri & (BLK - 1), jnp.int32)
            s = bd
            for c in range(D // L):
                fv = c * L + lane
                gu = plsc.load_gather(u_all, [srcu, fv, rcu])
                gi = plsc.load_gather(i_all, [srci, fv, rci])
                s = s + gu * gi * w_chunks[c]
            return lane_sum(s)

        NBUF = 6
        n_iters = (b_per_w + NBUF - 1) // NBUF
        for par in range(NBUF):
            fire(par, par)

        def ring_body(t, carry):
            e0 = NBUF * t
            for par in range(NBUF):
                e = jnp.minimum(e0 + par, b_per_w - 1)
                drain(par)
                v = element_value(e, par)
                fire(jnp.minimum(e0 + par + NBUF, b_per_w - 1), par)
                plsc.store_scatter(out_v, [jnp.full((L,), e, jnp.int32)], v,
                                   mask=lane == 0)
            return carry

        lax.fori_loop(0, n_iters, ring_body, 0)
        for par in range(NBUF):
            drain(par)

        pltpu.sync_copy(out_v, out_hbm.at[pl.ds(base, b_per_w)])

    return neumf_kernel


def kernel(users_index, items_index, user_mf_emb, item_mf_emb, W_pred, b_pred):
    ut = user_mf_emb.T
    it = item_mf_emb.T
    uedge = jnp.pad(user_mf_emb[EDGE0:].T, ((0, 0), (0, BLK - (NROWS - EDGE0))))
    iedge = jnp.pad(item_mf_emb[EDGE0:].T, ((0, 0), (0, BLK - (NROWS - EDGE0))))
    w_flat = W_pred.reshape(D)
    b_lane = jnp.full((L,), b_pred[0] / L, dtype=jnp.float32)
    call = _build_sc_call()
    out = call(users_index.astype(jnp.int32), items_index.astype(jnp.int32),
               ut, it, uedge, iedge, w_flat, b_lane)
    return out.reshape(BATCH, 1)
